# Initial kernel scaffold; baseline (speedup 1.0000x reference)
import jax
import jax.numpy as jnp
from jax.experimental import pallas as pl


def kernel(x, genres, W0):
    raise NotImplementedError("write your pallas kernel here")


# speedup vs baseline: 1.1120x; 1.1120x over previous
import functools

import jax
import jax.numpy as jnp
from jax import lax
from jax.experimental import pallas as pl
from jax.experimental.pallas import tpu as pltpu
from jax.experimental.pallas import tpu_sc as plsc

B = 1024
IN = 128
OUT = 128
E = 64

N_COLS = E * OUT
COL_BLK = 1024
N_STEPS = N_COLS // COL_BLK

NC = 2
NS = 16
NW = NC * NS
B_PER_W = B // NW
N_ROWS = 3 * B_PER_W


def _tc_matmul_body(x_ref, gp_ref, wt_ref, y_ref, xs_ref):
    i = pl.program_id(0)

    @pl.when(i == 0)
    def _():
        g = gp_ref[...]
        cnt = jnp.sum((g != 0).astype(jnp.float32), axis=1, keepdims=True)
        recip = 1.0 / jnp.maximum(cnt, 1.0)
        xs_ref[...] = x_ref[...] * recip

    y = lax.dot_general(
        xs_ref[...],
        wt_ref[...],
        (((1,), (0,)), ((), ())),
        preferred_element_type=jnp.float32,
        precision=lax.Precision.HIGHEST,
    )
    col = lax.broadcasted_iota(jnp.int32, (B, COL_BLK), 1)
    y_ref[...] = jnp.where((i == 0) & (col < OUT), 0.0, y)


def _tc_all_expert_matmul(x, genres_pad, wt):
    return pl.pallas_call(
        _tc_matmul_body,
        grid=(N_STEPS,),
        in_specs=[
            pl.BlockSpec((B, IN), lambda i: (0, 0)),
            pl.BlockSpec((B, 128), lambda i: (0, 0)),
            pl.BlockSpec((IN, COL_BLK), lambda i: (0, i)),
        ],
        out_specs=pl.BlockSpec((B, COL_BLK), lambda i: (0, i)),
        out_shape=jax.ShapeDtypeStruct((B, N_COLS), jnp.float32),
        scratch_shapes=[pltpu.VMEM((B, IN), jnp.float32)],
    )(x, genres_pad, wt)


def _sc_combine_body(gt_ref, y_ref, out_ref, gvm, idxv, rows, outv, sem):
    wid = lax.axis_index("s") * NC + lax.axis_index("c")
    base = wid * B_PER_W

    for j in range(3):
        pltpu.sync_copy(gt_ref.at[pl.ds(j * B + base, B_PER_W)], gvm.at[j])

    for j in range(3):
        for k in range(B_PER_W // 16):
            g = gvm[j, pl.ds(k * 16, 16)]
            t = lax.iota(jnp.int32, 16) + (base + k * 16)
            idxv[pl.ds(j * B_PER_W + k * 16, 16)] = t * E + g

    pltpu.async_copy(y_ref.at[idxv], rows, sem).wait()

    def body(t, carry):
        for c in range(OUT // 16):
            sl = pl.ds(c * 16, 16)
            s = rows[t, sl] + rows[t + B_PER_W, sl] + rows[t + 2 * B_PER_W, sl]
            outv[t, sl] = jnp.where(s >= 0.0, s, 0.01 * s)
        return carry

    lax.fori_loop(0, B_PER_W, body, 0)

    pltpu.sync_copy(outv, out_ref.at[pl.ds(base, B_PER_W)])


def _sc_combine(genres_t, y_table):
    mesh = plsc.VectorSubcoreMesh(core_axis_name="c", subcore_axis_name="s")
    run = pl.kernel(
        _sc_combine_body,
        out_type=jax.ShapeDtypeStruct((B, OUT), jnp.float32),
        mesh=mesh,
        scratch_types=[
            pltpu.VMEM((3, B_PER_W), jnp.int32),
            pltpu.VMEM((N_ROWS,), jnp.int32),
            pltpu.VMEM((N_ROWS, OUT), jnp.float32),
            pltpu.VMEM((B_PER_W, OUT), jnp.float32),
            pltpu.SemaphoreType.DMA,
        ],
    )
    return run(genres_t, y_table)


@jax.jit
def kernel(x, genres, W0):
    genres = genres.astype(jnp.int32)
    wt = jnp.transpose(W0, (1, 0, 2)).reshape(IN, N_COLS)
    genres_pad = jnp.pad(genres, ((0, 0), (0, 128 - genres.shape[1])))
    genres_t = genres.T.reshape(3 * B)

    y = _tc_all_expert_matmul(x, genres_pad, wt)
    y_table = y.reshape(B * E, OUT)
    return _sc_combine(genres_t, y_table)


# speedup vs baseline: 1.3937x; 1.2533x over previous
import functools

import jax
import jax.numpy as jnp
from jax import lax
from jax.experimental import pallas as pl
from jax.experimental.pallas import tpu as pltpu
from jax.experimental.pallas import tpu_sc as plsc

B = 1024
IN = 128
OUT = 128
E = 64

NC = 2
NS = 16
NW = NC * NS
B_PER_W = B // NW
N_ROWS = 3 * B_PER_W


def _tc_matmul_body(x_ref, gp_ref, w_ref, y_ref, xs_ref):
    e = pl.program_id(0)

    @pl.when(e == 0)
    def _():
        g = gp_ref[...]
        cnt = jnp.sum((g != 0).astype(jnp.float32), axis=1, keepdims=True)
        recip = 1.0 / jnp.maximum(cnt, 1.0)
        xs_ref[...] = x_ref[...] * recip

    y = lax.dot_general(
        xs_ref[...],
        w_ref[0],
        (((1,), (0,)), ((), ())),
        preferred_element_type=jnp.float32,
        precision=lax.Precision.HIGHEST,
    )
    y_ref[...] = jnp.where(e == 0, 0.0, y)


def _tc_all_expert_matmul(x, genres_pad, w0):
    return pl.pallas_call(
        _tc_matmul_body,
        grid=(E,),
        in_specs=[
            pl.BlockSpec((B, IN), lambda e: (0, 0)),
            pl.BlockSpec((B, 128), lambda e: (0, 0)),
            pl.BlockSpec((1, IN, OUT), lambda e: (e, 0, 0)),
        ],
        out_specs=pl.BlockSpec((B, OUT), lambda e: (e, 0)),
        out_shape=jax.ShapeDtypeStruct((E * B, OUT), jnp.float32),
        scratch_shapes=[pltpu.VMEM((B, IN), jnp.float32)],
    )(x, genres_pad, w0)


def _sc_combine_body(gt_ref, y_ref, out_ref, gvm, idxv, rows, outv, sem):
    wid = lax.axis_index("s") * NC + lax.axis_index("c")
    base = wid * B_PER_W

    for j in range(3):
        pltpu.sync_copy(gt_ref.at[pl.ds(j * B + base, B_PER_W)], gvm.at[j])

    for j in range(3):
        for k in range(B_PER_W // 16):
            g = gvm[j, pl.ds(k * 16, 16)]
            t = lax.iota(jnp.int32, 16) + (base + k * 16)
            idxv[pl.ds(j * B_PER_W + k * 16, 16)] = g * B + t

    pltpu.async_copy(y_ref.at[idxv], rows, sem).wait()

    def body(t, carry):
        for c in range(OUT // 16):
            sl = pl.ds(c * 16, 16)
            s = rows[t, sl] + rows[t + B_PER_W, sl] + rows[t + 2 * B_PER_W, sl]
            outv[t, sl] = jnp.where(s >= 0.0, s, 0.01 * s)
        return carry

    lax.fori_loop(0, B_PER_W, body, 0)

    pltpu.sync_copy(outv, out_ref.at[pl.ds(base, B_PER_W)])


def _sc_combine(genres_t, y_table):
    mesh = plsc.VectorSubcoreMesh(core_axis_name="c", subcore_axis_name="s")
    run = pl.kernel(
        _sc_combine_body,
        out_type=jax.ShapeDtypeStruct((B, OUT), jnp.float32),
        mesh=mesh,
        scratch_types=[
            pltpu.VMEM((3, B_PER_W), jnp.int32),
            pltpu.VMEM((N_ROWS,), jnp.int32),
            pltpu.VMEM((N_ROWS, OUT), jnp.float32),
            pltpu.VMEM((B_PER_W, OUT), jnp.float32),
            pltpu.SemaphoreType.DMA,
        ],
    )
    return run(genres_t, y_table)


@jax.jit
def kernel(x, genres, W0):
    genres = genres.astype(jnp.int32)
    genres_pad = jnp.pad(genres, ((0, 0), (0, 128 - genres.shape[1])))
    genres_t = genres.T.reshape(3 * B)

    y_table = _tc_all_expert_matmul(x, genres_pad, W0)
    return _sc_combine(genres_t, y_table)


# speedup vs baseline: 3.0155x; 2.1636x over previous
import functools

import jax
import jax.numpy as jnp
from jax import lax
from jax.experimental import pallas as pl
from jax.experimental.pallas import tpu as pltpu
from jax.experimental.pallas import tpu_sc as plsc

B = 1024
IN = 128
OUT = 128
E = 64

NC = 2
NS = 16
NW = NC * NS
B_PER_W = B // NW
C_PER_W = B_PER_W * E


def _sc_routing_body(gt_ref, c_ref, gvm, cvm):
    wid = lax.axis_index("s") * NC + lax.axis_index("c")
    base = wid * B_PER_W

    for j in range(3):
        pltpu.sync_copy(gt_ref.at[pl.ds(j * B + base, B_PER_W)], gvm.at[j])

    def zero_body(i, carry):
        cvm[pl.ds(i * 16, 16)] = jnp.zeros((16,), jnp.float32)
        return carry

    lax.fori_loop(0, C_PER_W // 16, zero_body, 0)

    for k in range(B_PER_W // 16):
        g = [gvm[j, pl.ds(k * 16, 16)] for j in range(3)]
        one = jnp.float32(1.0)
        zero = jnp.float32(0.0)
        cnt = sum(jnp.where(gj != 0, one, zero) for gj in g)
        recip = 1.0 / jnp.maximum(cnt, 1.0)
        tloc = lax.iota(jnp.int32, 16) + k * 16
        for j in range(3):
            val = jnp.where(g[j] != 0, recip, jnp.float32(0.0))
            plsc.addupdate_scatter(cvm, [tloc * E + g[j]], val)

    pltpu.sync_copy(cvm, c_ref.at[pl.ds(base * E, C_PER_W)])


def _sc_routing(genres_t):
    mesh = plsc.VectorSubcoreMesh(core_axis_name="c", subcore_axis_name="s")
    run = pl.kernel(
        _sc_routing_body,
        out_type=jax.ShapeDtypeStruct((B * E,), jnp.float32),
        mesh=mesh,
        scratch_types=[
            pltpu.VMEM((3, B_PER_W), jnp.int32),
            pltpu.VMEM((C_PER_W,), jnp.float32),
        ],
        compiler_params=pltpu.CompilerParams(needs_layout_passes=False),
    )
    return run(genres_t)


def _tc_matmul_body(x_ref, c_ref, w_ref, out_ref, xs_ref):
    x = x_ref[...]
    for e in range(E):
        col = c_ref[:, e : e + 1]
        xs_ref[:, e * IN : (e + 1) * IN] = (x * col).astype(jnp.bfloat16)
    acc = lax.dot_general(
        xs_ref[...],
        w_ref[...],
        (((1,), (0,)), ((), ())),
        preferred_element_type=jnp.float32,
    )
    out_ref[...] = jnp.where(acc >= 0.0, acc, 0.01 * acc)


def _tc_combine_matmul(x, c_mat, w_flat):
    return pl.pallas_call(
        _tc_matmul_body,
        grid=(1,),
        in_specs=[
            pl.BlockSpec((B, IN), lambda i: (0, 0)),
            pl.BlockSpec((B, E), lambda i: (0, 0)),
            pl.BlockSpec((E * IN, OUT), lambda i: (0, 0)),
        ],
        out_specs=pl.BlockSpec((B, OUT), lambda i: (0, 0)),
        out_shape=jax.ShapeDtypeStruct((B, OUT), jnp.float32),
        scratch_shapes=[pltpu.VMEM((B, E * IN), jnp.bfloat16)],
    )(x, c_mat, w_flat)


@jax.jit
def kernel(x, genres, W0):
    genres = genres.astype(jnp.int32)
    genres_t = genres.T.reshape(3 * B)
    w_flat = W0.reshape(E * IN, OUT).astype(jnp.bfloat16)

    c_mat = _sc_routing(genres_t).reshape(B, E)
    return _tc_combine_matmul(x, c_mat, w_flat)


# speedup vs baseline: 3.0356x; 1.0067x over previous
import functools

import jax
import jax.numpy as jnp
from jax import lax
from jax.experimental import pallas as pl
from jax.experimental.pallas import tpu as pltpu
from jax.experimental.pallas import tpu_sc as plsc

B = 1024
IN = 128
OUT = 128
E = 64

NC = 2
NS = 16
NW = NC * NS
B_PER_W = B // NW
C_PER_W = B_PER_W * E


def _sc_routing_body(gt_ref, c_ref, gvm, cvm):
    wid = lax.axis_index("s") * NC + lax.axis_index("c")
    base = wid * B_PER_W

    pltpu.sync_copy(gt_ref.at[pl.ds(base * 3, B_PER_W * 3)], gvm)

    def zero_body(i, carry):
        cvm[pl.ds(i * 16, 16)] = jnp.zeros((16,), jnp.float32)
        return carry

    lax.fori_loop(0, C_PER_W // 16, zero_body, 0)

    for k in range(B_PER_W // 16):
        tl = lax.iota(jnp.int32, 16) + k * 16
        g = [plsc.load_gather(gvm, [tl * 3 + j]) for j in range(3)]
        one = jnp.float32(1.0)
        zero = jnp.float32(0.0)
        cnt = sum(jnp.where(gj != 0, one, zero) for gj in g)
        recip = 1.0 / jnp.maximum(cnt, 1.0)
        tloc = lax.iota(jnp.int32, 16) + k * 16
        for j in range(3):
            val = jnp.where(g[j] != 0, recip, jnp.float32(0.0))
            plsc.addupdate_scatter(cvm, [tloc * E + g[j]], val)

    pltpu.sync_copy(cvm, c_ref.at[pl.ds(base * E, C_PER_W)])


def _sc_routing(genres_t):
    mesh = plsc.VectorSubcoreMesh(core_axis_name="c", subcore_axis_name="s")
    run = pl.kernel(
        _sc_routing_body,
        out_type=jax.ShapeDtypeStruct((B * E,), jnp.float32),
        mesh=mesh,
        scratch_types=[
            pltpu.VMEM((B_PER_W * 3,), jnp.int32),
            pltpu.VMEM((C_PER_W,), jnp.float32),
        ],
        compiler_params=pltpu.CompilerParams(needs_layout_passes=False),
    )
    return run(genres_t)


def _tc_matmul_body(x_ref, c_ref, w_ref, out_ref, xs_ref, wb_ref):
    x = x_ref[...]
    for e in range(E):
        col = c_ref[:, e : e + 1]
        xs_ref[:, e * IN : (e + 1) * IN] = (x * col).astype(jnp.bfloat16)
    wb_ref[...] = w_ref[...].astype(jnp.bfloat16)
    acc = lax.dot_general(
        xs_ref[...],
        wb_ref[...],
        (((1,), (0,)), ((), ())),
        preferred_element_type=jnp.float32,
    )
    out_ref[...] = jnp.where(acc >= 0.0, acc, 0.01 * acc)


def _tc_combine_matmul(x, c_mat, w_flat):
    return pl.pallas_call(
        _tc_matmul_body,
        grid=(1,),
        in_specs=[
            pl.BlockSpec((B, IN), lambda i: (0, 0)),
            pl.BlockSpec((B, E), lambda i: (0, 0)),
            pl.BlockSpec((E * IN, OUT), lambda i: (0, 0)),
        ],
        out_specs=pl.BlockSpec((B, OUT), lambda i: (0, 0)),
        out_shape=jax.ShapeDtypeStruct((B, OUT), jnp.float32),
        scratch_shapes=[
            pltpu.VMEM((B, E * IN), jnp.bfloat16),
            pltpu.VMEM((E * IN, OUT), jnp.bfloat16),
        ],
    )(x, c_mat, w_flat)


@jax.jit
def kernel(x, genres, W0):
    genres = genres.astype(jnp.int32)
    genres_t = genres.reshape(B * 3)
    w_flat = W0.reshape(E * IN, OUT)

    c_mat = _sc_routing(genres_t).reshape(B, E)
    return _tc_combine_matmul(x, c_mat, w_flat)


# speedup vs baseline: 3.1425x; 1.0352x over previous
import functools

import jax
import jax.numpy as jnp
from jax import lax
from jax.experimental import pallas as pl
from jax.experimental.pallas import tpu as pltpu
from jax.experimental.pallas import tpu_sc as plsc

B = 1024
IN = 128
OUT = 128
E = 64
CP = 128

NC = 2
NS = 16
NW = NC * NS
B_PER_W = B // NW
C_PER_W = B_PER_W * CP

E_BLK = 8
N_STEPS = E // E_BLK


def _sc_routing_body(gt_ref, c_ref, gvm, cvm):
    wid = lax.axis_index("s") * NC + lax.axis_index("c")
    base = wid * B_PER_W

    pltpu.sync_copy(gt_ref.at[pl.ds(base * 3, B_PER_W * 3)], gvm)

    def zero_body(i, carry):
        cvm[pl.ds(i * 16, 16)] = jnp.zeros((16,), jnp.float32)
        return carry

    lax.fori_loop(0, C_PER_W // 16, zero_body, 0)

    for k in range(B_PER_W // 16):
        tl = lax.iota(jnp.int32, 16) + k * 16
        g = [plsc.load_gather(gvm, [tl * 3 + j]) for j in range(3)]
        one = jnp.float32(1.0)
        zero = jnp.float32(0.0)
        cnt = sum(jnp.where(gj != 0, one, zero) for gj in g)
        recip = 1.0 / jnp.maximum(cnt, 1.0)
        for j in range(3):
            val = jnp.where(g[j] != 0, recip, zero)
            plsc.addupdate_scatter(cvm, [tl * CP + g[j]], val)

    pltpu.sync_copy(cvm, c_ref.at[pl.ds(base * CP, C_PER_W)])


def _sc_routing(genres_t):
    mesh = plsc.VectorSubcoreMesh(core_axis_name="c", subcore_axis_name="s")
    run = pl.kernel(
        _sc_routing_body,
        out_type=jax.ShapeDtypeStruct((B * CP,), jnp.float32),
        mesh=mesh,
        scratch_types=[
            pltpu.VMEM((B_PER_W * 3,), jnp.int32),
            pltpu.VMEM((C_PER_W,), jnp.float32),
        ],
        compiler_params=pltpu.CompilerParams(needs_layout_passes=False),
    )
    return run(genres_t)


def _tc_matmul_body(x_ref, c_ref, w_ref, out_ref, xs_ref, wb_ref):
    x = x_ref[...]
    for e in range(E):
        col = c_ref[:, e : e + 1]
        xs_ref[:, e * IN : (e + 1) * IN] = (x * col).astype(jnp.bfloat16)
    wb_ref[...] = w_ref[...].astype(jnp.bfloat16)
    acc = lax.dot_general(
        xs_ref[...],
        wb_ref[...],
        (((1,), (0,)), ((), ())),
        preferred_element_type=jnp.float32,
    )
    out_ref[...] = jnp.where(acc >= 0.0, acc, 0.01 * acc)


def _tc_combine_matmul(x, c_mat, w_flat):
    return pl.pallas_call(
        _tc_matmul_body,
        grid=(1,),
        in_specs=[
            pl.BlockSpec((B, IN), lambda i: (0, 0)),
            pl.BlockSpec((B, CP), lambda i: (0, 0)),
            pl.BlockSpec((E * IN, OUT), lambda i: (0, 0)),
        ],
        out_specs=pl.BlockSpec((B, OUT), lambda i: (0, 0)),
        out_shape=jax.ShapeDtypeStruct((B, OUT), jnp.float32),
        scratch_shapes=[
            pltpu.VMEM((B, E * IN), jnp.bfloat16),
            pltpu.VMEM((E * IN, OUT), jnp.bfloat16),
        ],
    )(x, c_mat, w_flat)


@jax.jit
def kernel(x, genres, W0):
    genres = genres.astype(jnp.int32)
    genres_t = genres.reshape(B * 3)
    w_flat = W0.reshape(E * IN, OUT)

    c_mat = _sc_routing(genres_t).reshape(B, CP)
    return _tc_combine_matmul(x, c_mat, w_flat)


# speedup vs baseline: 3.3910x; 1.0791x over previous
import functools

import jax
import jax.numpy as jnp
from jax import lax
from jax.experimental import pallas as pl
from jax.experimental.pallas import tpu as pltpu
from jax.experimental.pallas import tpu_sc as plsc

B = 1024
IN = 128
OUT = 128
E = 64
CP = 128

NC = 2
NS = 16
NW = NC * NS
B_PER_W = B // NW
C_PER_W = B_PER_W * CP

E_BLK = 8
N_STEPS = E // E_BLK


def _sc_routing_body(gt_ref, c_ref, gvm, cvm):
    wid = lax.axis_index("s") * NC + lax.axis_index("c")
    base = wid * B_PER_W

    pltpu.sync_copy(gt_ref.at[pl.ds(base * 3, B_PER_W * 3)], gvm)

    def zero_body(i, carry):
        cvm[pl.ds(i * 16, 16)] = jnp.zeros((16,), jnp.float32)
        return carry

    lax.fori_loop(0, C_PER_W // 16, zero_body, 0)

    for k in range(B_PER_W // 16):
        tl = lax.iota(jnp.int32, 16) + k * 16
        g = [plsc.load_gather(gvm, [tl * 3 + j]) for j in range(3)]
        one = jnp.float32(1.0)
        zero = jnp.float32(0.0)
        cnt = sum(jnp.where(gj != 0, one, zero) for gj in g)
        recip = 1.0 / jnp.maximum(cnt, 1.0)
        for j in range(3):
            val = jnp.where(g[j] != 0, recip, zero)
            plsc.addupdate_scatter(cvm, [tl * CP + g[j]], val)

    pltpu.sync_copy(cvm, c_ref.at[pl.ds(base * CP, C_PER_W)])


def _sc_routing(genres_t):
    mesh = plsc.VectorSubcoreMesh(core_axis_name="c", subcore_axis_name="s")
    run = pl.kernel(
        _sc_routing_body,
        out_type=jax.ShapeDtypeStruct((B * CP,), jnp.float32),
        mesh=mesh,
        scratch_types=[
            pltpu.VMEM((B_PER_W * 3,), jnp.int32),
            pltpu.VMEM((C_PER_W,), jnp.float32),
        ],
        compiler_params=pltpu.CompilerParams(needs_layout_passes=False),
    )
    return run(genres_t)


def _tc_matmul_body(x_ref, c_ref, w_ref, out_ref, xs_ref, wb_ref):
    x = x_ref[...].astype(jnp.bfloat16)
    c = c_ref[...].astype(jnp.bfloat16)
    for e in range(E):
        col = c[:, e : e + 1]
        xs_ref[:, e * IN : (e + 1) * IN] = x * col
    wb_ref[...] = w_ref[...].astype(jnp.bfloat16)
    acc = lax.dot_general(
        xs_ref[...],
        wb_ref[...],
        (((1,), (0,)), ((), ())),
        preferred_element_type=jnp.float32,
    )
    out_ref[...] = jnp.where(acc >= 0.0, acc, 0.01 * acc)


def _tc_combine_matmul(x, c_mat, w_flat):
    return pl.pallas_call(
        _tc_matmul_body,
        grid=(1,),
        in_specs=[
            pl.BlockSpec((B, IN), lambda i: (0, 0)),
            pl.BlockSpec((B, CP), lambda i: (0, 0)),
            pl.BlockSpec((E * IN, OUT), lambda i: (0, 0)),
        ],
        out_specs=pl.BlockSpec((B, OUT), lambda i: (0, 0)),
        out_shape=jax.ShapeDtypeStruct((B, OUT), jnp.float32),
        scratch_shapes=[
            pltpu.VMEM((B, E * IN), jnp.bfloat16),
            pltpu.VMEM((E * IN, OUT), jnp.bfloat16),
        ],
    )(x, c_mat, w_flat)


@jax.jit
def kernel(x, genres, W0):
    genres = genres.astype(jnp.int32)
    genres_t = genres.reshape(B * 3)
    w_flat = W0.reshape(E * IN, OUT)

    c_mat = _sc_routing(genres_t).reshape(B, CP)
    return _tc_combine_matmul(x, c_mat, w_flat)
